# baseline (speedup 1.0000x reference)
import jax
import jax.numpy as jnp
from jax.experimental import pallas as pl
from jax.experimental.pallas import tpu as pltpu

BM1 = 128
BM2 = 512


def _supports_kernel(x_ref, w_ref, o_ref):
    s = jnp.dot(x_ref[...], w_ref[...], preferred_element_type=jnp.float32)
    h = o_ref.shape[2]
    for i in range(3):
        o_ref[i] = s[:, i * h:(i + 1) * h]


def _layer1_kernel(a_ref, s_ref, bsum_ref, w2_ref, h_ref, g_ref):
    acc = bsum_ref[...]
    for i in range(3):
        acc = acc + jnp.dot(a_ref[i].astype(jnp.bfloat16),
                            s_ref[i].astype(jnp.bfloat16),
                            preferred_element_type=jnp.float32)
    h = jnp.maximum(acc, 0.0)
    h_ref[...] = h
    g_ref[...] = jnp.dot(h, w2_ref[...], preferred_element_type=jnp.float32)


def _layer2_kernel(a_ref, g_ref, b2_ref, o_ref):
    z = jnp.dot(a_ref[...].astype(jnp.bfloat16),
                g_ref[...].astype(jnp.bfloat16),
                preferred_element_type=jnp.float32)
    z = z + b2_ref[...]
    m = jnp.max(z, axis=1, keepdims=True)
    e = jnp.exp(z - m)
    lse = m + jnp.log(jnp.sum(e, axis=1, keepdims=True))
    o_ref[...] = z - lse


@jax.jit
def kernel(x, adj, sub_adj, W1_1, b1_1, W1_2, b1_2, W1_3, b1_3, W2, b2):
    n, f = x.shape
    h = W1_1.shape[1]
    c = W2.shape[1]

    wcat = jnp.concatenate([W1_1, W1_2, W1_3], axis=1)
    bsum = (b1_1 + b1_2 + b1_3).reshape(1, h)
    b2r = b2.reshape(1, c)

    s = pl.pallas_call(
        _supports_kernel,
        grid=(pl.cdiv(n, 2000),),
        in_specs=[
            pl.BlockSpec((2000, f), lambda i: (i, 0)),
            pl.BlockSpec((f, 3 * h), lambda i: (0, 0)),
        ],
        out_specs=pl.BlockSpec((3, 2000, h), lambda i: (0, i, 0)),
        out_shape=jax.ShapeDtypeStruct((3, n, h), jnp.float32),
        compiler_params=pltpu.CompilerParams(
            dimension_semantics=("arbitrary",)),
    )(x, wcat)

    _, g = pl.pallas_call(
        _layer1_kernel,
        grid=(pl.cdiv(n, BM1),),
        in_specs=[
            pl.BlockSpec((3, BM1, n), lambda i: (0, i, 0)),
            pl.BlockSpec((3, n, h), lambda i: (0, 0, 0)),
            pl.BlockSpec((1, h), lambda i: (0, 0)),
            pl.BlockSpec((f, c), lambda i: (0, 0)),
        ],
        out_specs=[
            pl.BlockSpec((BM1, h), lambda i: (i, 0)),
            pl.BlockSpec((BM1, c), lambda i: (i, 0)),
        ],
        out_shape=[
            jax.ShapeDtypeStruct((n, h), jnp.float32),
            jax.ShapeDtypeStruct((n, c), jnp.float32),
        ],
        compiler_params=pltpu.CompilerParams(
            dimension_semantics=("arbitrary",)),
    )(sub_adj, s, bsum, W2)

    out = pl.pallas_call(
        _layer2_kernel,
        grid=(pl.cdiv(n, BM2),),
        in_specs=[
            pl.BlockSpec((BM2, n), lambda i: (i, 0)),
            pl.BlockSpec((n, c), lambda i: (0, 0)),
            pl.BlockSpec((1, c), lambda i: (0, 0)),
        ],
        out_specs=pl.BlockSpec((BM2, c), lambda i: (i, 0)),
        out_shape=jax.ShapeDtypeStruct((n, c), jnp.float32),
        compiler_params=pltpu.CompilerParams(
            dimension_semantics=("arbitrary",)),
    )(adj, g, b2r)

    return out


# speedup vs baseline: 1.0251x; 1.0251x over previous
import jax
import jax.numpy as jnp
from jax.experimental import pallas as pl
from jax.experimental.pallas import tpu as pltpu

BM1 = 128
BM2 = 512


def _layer1_kernel(a_ref, x_ref, wcat_ref, bsum_ref, w2_ref, g_ref, s_ref):
    h = w2_ref.shape[0]

    @pl.when(pl.program_id(0) == 0)
    def _():
        x = x_ref[...]
        for i in range(3):
            s_ref[i] = jnp.dot(x, wcat_ref[:, i * h:(i + 1) * h],
                               preferred_element_type=jnp.float32)

    acc = bsum_ref[...]
    for i in range(3):
        acc = acc + jnp.dot(a_ref[i], s_ref[i],
                            preferred_element_type=jnp.float32)
    hid = jnp.maximum(acc, 0.0)
    g_ref[...] = jnp.dot(hid, w2_ref[...], preferred_element_type=jnp.float32)


def _layer2_kernel(a_ref, g_ref, b2_ref, o_ref):
    z = jnp.dot(a_ref[...], g_ref[...], preferred_element_type=jnp.float32)
    z = z + b2_ref[...]
    m = jnp.max(z, axis=1, keepdims=True)
    e = jnp.exp(z - m)
    lse = m + jnp.log(jnp.sum(e, axis=1, keepdims=True))
    o_ref[...] = z - lse


@jax.jit
def kernel(x, adj, sub_adj, W1_1, b1_1, W1_2, b1_2, W1_3, b1_3, W2, b2):
    n, f = x.shape
    h = W1_1.shape[1]
    c = W2.shape[1]

    wcat = jnp.concatenate([W1_1, W1_2, W1_3], axis=1)
    bsum = (b1_1 + b1_2 + b1_3).reshape(1, h)
    b2r = b2.reshape(1, c)

    g = pl.pallas_call(
        _layer1_kernel,
        grid=(pl.cdiv(n, BM1),),
        in_specs=[
            pl.BlockSpec((3, BM1, n), lambda i: (0, i, 0)),
            pl.BlockSpec((n, f), lambda i: (0, 0)),
            pl.BlockSpec((f, 3 * h), lambda i: (0, 0)),
            pl.BlockSpec((1, h), lambda i: (0, 0)),
            pl.BlockSpec((h, c), lambda i: (0, 0)),
        ],
        out_specs=pl.BlockSpec((BM1, c), lambda i: (i, 0)),
        out_shape=jax.ShapeDtypeStruct((n, c), jnp.float32),
        scratch_shapes=[pltpu.VMEM((3, n, h), jnp.float32)],
        compiler_params=pltpu.CompilerParams(
            dimension_semantics=("arbitrary",)),
    )(sub_adj, x, wcat, bsum, W2)

    out = pl.pallas_call(
        _layer2_kernel,
        grid=(pl.cdiv(n, BM2),),
        in_specs=[
            pl.BlockSpec((BM2, n), lambda i: (i, 0)),
            pl.BlockSpec((n, c), lambda i: (0, 0)),
            pl.BlockSpec((1, c), lambda i: (0, 0)),
        ],
        out_specs=pl.BlockSpec((BM2, c), lambda i: (i, 0)),
        out_shape=jax.ShapeDtypeStruct((n, c), jnp.float32),
        compiler_params=pltpu.CompilerParams(
            dimension_semantics=("arbitrary",)),
    )(adj, g, b2r)

    return out
